# Initial kernel scaffold; baseline (speedup 1.0000x reference)
import jax
import jax.numpy as jnp
from jax.experimental import pallas as pl


def kernel(x, W1, g1, b1, W2, g2, b2, W3, g3, b3, W4, g4, b4, W5, g5, b5, Wc1, bc1, gc1, bec1, Wc2, bc2, gc2, bec2, Wc3, bc3):
    raise NotImplementedError("write your pallas kernel here")


# speedup vs baseline: 11.7601x; 11.7601x over previous
import functools

import jax
import jax.numpy as jnp
from jax import lax
from jax.experimental import pallas as pl
from jax.experimental.pallas import tpu as pltpu
from jax.experimental.pallas import tpu_sc as plsc

_B = 16
_N = 1024
_K = 20
_EPS = 1e-5
_NEG = -3.0e38


def _knn_kernel(x_ref, idx_ref, tidx_ref, *, n, dp):
    b = pl.program_id(0)
    x = x_ref[...]
    xx = jnp.sum(x * x, axis=1, keepdims=True)
    xb = x.astype(jnp.bfloat16)
    g = lax.dot_general(xb, xb, (((1,), (1,)), ((), ())),
                        preferred_element_type=jnp.float32)
    inner = -2.0 * g
    xxr = jnp.transpose(xx)
    score = ((-xxr) - inner) - xx
    lane = lax.broadcasted_iota(jnp.int32, (n, n), 1)
    cur = score
    for i in range(_K):
        m = jnp.max(cur, axis=1, keepdims=True)
        cand = jnp.where(cur == m, lane, n + 1)
        am = jnp.min(cand, axis=1, keepdims=True)
        tidx_ref[:, pl.ds(i, 1)] = am
        cur = jnp.where(lane == am, _NEG, cur)
    t = jnp.transpose(tidx_ref[...])
    idx_ref[0] = t[:_K, :] + b * n


def _knn_topk(x, dp):
    return pl.pallas_call(
        functools.partial(_knn_kernel, n=_N, dp=dp),
        grid=(_B,),
        in_specs=[pl.BlockSpec((_N, dp), lambda b: (b, 0))],
        out_specs=pl.BlockSpec((1, _K, _N), lambda b: (b, 0, 0)),
        out_shape=jax.ShapeDtypeStruct((_B, _K, _N), jnp.int32),
        scratch_shapes=[pltpu.VMEM((_N, 32), jnp.int32)],
    )(x)


def _gather_rows(table, idx_flat, dp):
    m = idx_flat.shape[0]
    info = plsc.get_sparse_core_info()
    nw = info.num_cores * info.num_subcores
    chunk = m // nw
    win = 128
    nwin = chunk // win

    @functools.partial(
        pl.kernel,
        mesh=plsc.VectorSubcoreMesh(core_axis_name="c", subcore_axis_name="s"),
        out_type=jax.ShapeDtypeStruct((m, dp), jnp.float32),
        scratch_types=[
            pltpu.VMEM((win,), jnp.int32),
            pltpu.VMEM((win, dp), jnp.float32),
            pltpu.SemaphoreType.DMA,
        ],
    )
    def gk(table_hbm, idx_hbm, out_hbm, idx_v, rows_v, sem):
        wid = lax.axis_index("s") * info.num_cores + lax.axis_index("c")
        base = wid * chunk

        def body(w, carry):
            off = base + w * win
            pltpu.sync_copy(idx_hbm.at[pl.ds(off, win)], idx_v)
            pltpu.async_copy(table_hbm.at[idx_v], rows_v, sem).wait()
            pltpu.sync_copy(rows_v, out_hbm.at[pl.ds(off, win)])
            return carry

        lax.fori_loop(0, nwin, body, 0)

    return gk(table, idx_flat)


def _conv_kernel(g_ref, x_ref, w_ref, s_ref, bt_ref, out_ref, *, ch, dp, o):
    xb = x_ref[...]
    g = g_ref[0]
    diff = g - xb
    xbk = jnp.broadcast_to(xb, (_K, ch, dp))
    f = jnp.concatenate([diff, xbk], axis=2)
    f2 = f.reshape(_K * ch, 2 * dp).astype(jnp.bfloat16)
    w = w_ref[...].astype(jnp.bfloat16)
    y = lax.dot_general(f2, w, (((1,), (0,)), ((), ())),
                        preferred_element_type=jnp.float32)
    ymax = y[0:ch, :]
    ymin = y[0:ch, :]
    for k in range(1, _K):
        blk = y[k * ch:(k + 1) * ch, :]
        ymax = jnp.maximum(ymax, blk)
        ymin = jnp.minimum(ymin, blk)
    s = s_ref[...]
    bt = bt_ref[...]

    def act(v):
        t = v * s + bt
        return jnp.where(t >= 0, t, 0.2 * t)

    out_ref[...] = jnp.maximum(act(ymax), act(ymin))


def _edge_conv(gath, x, wpad_t, scale, beta, dp, o, ch=256):
    nb = _N // ch
    return pl.pallas_call(
        functools.partial(_conv_kernel, ch=ch, dp=dp, o=o),
        grid=(_B, nb),
        in_specs=[
            pl.BlockSpec((1, _K, ch, dp), lambda b, j: (b, 0, j, 0)),
            pl.BlockSpec((1, ch, dp), lambda b, j: (b * nb + j, 0, 0)),
            pl.BlockSpec((2 * dp, o), lambda b, j: (0, 0)),
            pl.BlockSpec((1, o), lambda b, j: (0, 0)),
            pl.BlockSpec((1, o), lambda b, j: (0, 0)),
        ],
        out_specs=pl.BlockSpec((ch, o), lambda b, j: (b * nb + j, 0)),
        out_shape=jax.ShapeDtypeStruct((_B * _N, o), jnp.float32),
    )(gath, x.reshape(_B * nb, ch, dp), wpad_t, scale, beta)


def _pool_kernel(x1_ref, x2_ref, x3_ref, x4_ref, w_ref, s_ref, bt_ref,
                 out_ref):
    cat = jnp.concatenate(
        [x1_ref[...], x2_ref[...], x3_ref[...], x4_ref[...]], axis=1)
    y = lax.dot_general(cat.astype(jnp.bfloat16),
                        w_ref[...].astype(jnp.bfloat16),
                        (((1,), (0,)), ((), ())),
                        preferred_element_type=jnp.float32)
    t = y * s_ref[...] + bt_ref[...]
    z = jnp.where(t >= 0, t, 0.2 * t)
    out_ref[0] = jnp.max(z, axis=0, keepdims=True)


def _pool(x1, x2, x3, x4, w5_t, scale5, beta5):
    return pl.pallas_call(
        _pool_kernel,
        grid=(_B,),
        in_specs=[
            pl.BlockSpec((_N, 64), lambda b: (b, 0)),
            pl.BlockSpec((_N, 64), lambda b: (b, 0)),
            pl.BlockSpec((_N, 128), lambda b: (b, 0)),
            pl.BlockSpec((_N, 256), lambda b: (b, 0)),
            pl.BlockSpec((512, 512), lambda b: (0, 0)),
            pl.BlockSpec((1, 512), lambda b: (0, 0)),
            pl.BlockSpec((1, 512), lambda b: (0, 0)),
        ],
        out_specs=pl.BlockSpec((1, 1, 512), lambda b: (b, 0, 0)),
        out_shape=jax.ShapeDtypeStruct((_B, 1, 512), jnp.float32),
    )(x1, x2, x3, x4, w5_t, scale5, beta5)


def _mlp_kernel(p_ref, w1_ref, b1_ref, s1_ref, e1_ref, w2_ref, b2_ref,
                s2_ref, e2_ref, w3_ref, b3_ref, out_ref):
    def mm(a, w):
        return lax.dot_general(a.astype(jnp.bfloat16),
                               w[...].astype(jnp.bfloat16),
                               (((1,), (0,)), ((), ())),
                               preferred_element_type=jnp.float32)

    y = mm(p_ref[...], w1_ref) + b1_ref[...]
    t = y * s1_ref[...] + e1_ref[...]
    y = jnp.where(t >= 0, t, 0.2 * t)
    y = mm(y, w2_ref) + b2_ref[...]
    t = y * s2_ref[...] + e2_ref[...]
    y = jnp.where(t >= 0, t, 0.2 * t)
    out_ref[...] = mm(y, w3_ref) + b3_ref[...]


def _mlp(pooled, wc1_t, bc1, sc1, ec1, wc2_t, bc2, sc2, ec2, wc3_t, bc3):
    args = (pooled, wc1_t, bc1, sc1, ec1, wc2_t, bc2, sc2, ec2, wc3_t, bc3)
    return pl.pallas_call(
        _mlp_kernel,
        in_specs=[pl.BlockSpec(a.shape, lambda nd=len(a.shape): (0,) * nd)
                  for a in args],
        out_specs=pl.BlockSpec((_B, 40), lambda: (0, 0)),
        out_shape=jax.ShapeDtypeStruct((_B, 40), jnp.float32),
    )(*args)


def _layer(x_rows, dp, wpad_t, scale, beta, o):
    idx = _knn_topk(x_rows, dp)
    rows = _gather_rows(x_rows, idx.reshape(-1), dp)
    gath = rows.reshape(_B, _K, _N, dp)
    return _edge_conv(gath, x_rows, wpad_t, scale, beta, dp, o)


def kernel(x, W1, g1, b1, W2, g2, b2, W3, g3, b3, W4, g4, b4, W5, g5, b5,
           Wc1, bc1, gc1, bec1, Wc2, bc2, gc2, bec2, Wc3, bc3):
    rsq = jnp.sqrt(jnp.float32(1.0) + _EPS)

    def row(v):
        return v.reshape(1, -1)

    def wpad(W, d):
        o = W.shape[0]
        wp = jnp.zeros((256, o), jnp.float32)
        return wp.at[0:d, :].set(W[:, 0:d].T).at[128:128 + d, :].set(
            W[:, d:2 * d].T)

    def pad128(v):
        return jnp.pad(v, ((0, 0), (0, 128 - v.shape[1])))

    x0p = pad128(x.reshape(_B * _N, 3))
    x1 = _layer(x0p, 128, wpad(W1, 3), row(g1 / rsq), row(b1), 64)
    x2 = _layer(pad128(x1), 128, wpad(W2, 64), row(g2 / rsq), row(b2), 64)
    x3 = _layer(pad128(x2), 128, wpad(W3, 64), row(g3 / rsq), row(b3), 128)
    x4 = _layer(x3, 128, W4.T, row(g4 / rsq), row(b4), 256)

    pooled = _pool(x1, x2, x3, x4, W5.T, row(g5 / rsq),
                   row(b5)).reshape(_B, 512)

    return _mlp(pooled, Wc1.T, row(bc1), row(gc1 / rsq), row(bec1),
                Wc2.T, row(bc2), row(gc2 / rsq), row(bec2),
                Wc3.T, row(bc3))


# speedup vs baseline: 14.8504x; 1.2628x over previous
import functools

import jax
import jax.numpy as jnp
from jax import lax
from jax.experimental import pallas as pl
from jax.experimental.pallas import tpu as pltpu
from jax.experimental.pallas import tpu_sc as plsc

_B = 16
_N = 1024
_K = 20
_EPS = 1e-5
_NEG = -3.0e38


def _knn_kernel(x_ref, idx_ref, tidx_ref, *, n, dp):
    b = pl.program_id(0)
    x = x_ref[...]
    xx = jnp.sum(x * x, axis=1, keepdims=True)
    xb = x.astype(jnp.bfloat16)
    g = lax.dot_general(xb, xb, (((1,), (1,)), ((), ())),
                        preferred_element_type=jnp.float32)
    inner = -2.0 * g
    xxr = jnp.transpose(xx)
    score = ((-xxr) - inner) - xx
    lane = lax.broadcasted_iota(jnp.int32, (n, n), 1)
    cur = score
    for i in range(_K):
        m = jnp.max(cur, axis=1, keepdims=True)
        cand = jnp.where(cur == m, lane, n + 1)
        am = jnp.min(cand, axis=1, keepdims=True)
        tidx_ref[:, pl.ds(i, 1)] = am
        cur = jnp.where(lane == am, _NEG, cur)
    t = jnp.transpose(tidx_ref[...])
    idx_ref[0] = t[:_K, :] + b * n


def _knn_topk(x, dp, nbat):
    return pl.pallas_call(
        functools.partial(_knn_kernel, n=_N, dp=dp),
        grid=(nbat,),
        in_specs=[pl.BlockSpec((_N, dp), lambda b: (b, 0))],
        out_specs=pl.BlockSpec((1, _K, _N), lambda b: (b, 0, 0)),
        out_shape=jax.ShapeDtypeStruct((nbat, _K, _N), jnp.int32),
        scratch_shapes=[pltpu.VMEM((_N, 32), jnp.int32)],
    )(x)


def _gather_rows(table, idx_flat, dp):
    m = idx_flat.shape[0]
    info = plsc.get_sparse_core_info()
    nw = info.num_cores * info.num_subcores
    chunk = m // nw
    win = 128
    nwin = chunk // win

    @functools.partial(
        pl.kernel,
        mesh=plsc.VectorSubcoreMesh(core_axis_name="c", subcore_axis_name="s"),
        out_type=jax.ShapeDtypeStruct((m, dp), jnp.float32),
        scratch_types=[
            pltpu.VMEM((2, win), jnp.int32),
            pltpu.VMEM((2, win, dp), jnp.float32),
            pltpu.SemaphoreType.DMA,
            pltpu.SemaphoreType.DMA,
        ],
    )
    def gk(table_hbm, idx_hbm, out_hbm, idx_v, rows_v, gsem, osem):
        wid = lax.axis_index("s") * info.num_cores + lax.axis_index("c")
        base = wid * chunk

        def body(w, carry):
            slot = lax.rem(w, 2)
            off = base + w * win
            pltpu.sync_copy(idx_hbm.at[pl.ds(off, win)], idx_v.at[slot])
            pltpu.async_copy(table_hbm.at[idx_v.at[slot]],
                             rows_v.at[slot], gsem).wait()
            pltpu.async_copy(rows_v.at[slot], out_hbm.at[pl.ds(off, win)],
                             osem)

            @pl.when(w >= 1)
            def _():
                pltpu.make_async_copy(
                    rows_v.at[slot], out_hbm.at[pl.ds(off, win)], osem
                ).wait()

            return carry

        lax.fori_loop(0, nwin, body, 0)
        pltpu.make_async_copy(rows_v.at[0],
                              out_hbm.at[pl.ds(base, win)], osem).wait()

    return gk(table, idx_flat)


def _conv_kernel(g_ref, x_ref, w_ref, s_ref, bt_ref, out_ref, *, ch, dp, o):
    xb = x_ref[...]
    g = g_ref[0]
    diff = g - xb
    xbk = jnp.broadcast_to(xb, (_K, ch, dp))
    f = jnp.concatenate([diff, xbk], axis=2)
    f2 = f.reshape(_K * ch, 2 * dp).astype(jnp.bfloat16)
    w = w_ref[...].astype(jnp.bfloat16)
    y = lax.dot_general(f2, w, (((1,), (0,)), ((), ())),
                        preferred_element_type=jnp.float32)
    ymax = y[0:ch, :]
    ymin = y[0:ch, :]
    for k in range(1, _K):
        blk = y[k * ch:(k + 1) * ch, :]
        ymax = jnp.maximum(ymax, blk)
        ymin = jnp.minimum(ymin, blk)
    s = s_ref[...]
    bt = bt_ref[...]

    def act(v):
        t = v * s + bt
        return jnp.where(t >= 0, t, 0.2 * t)

    out_ref[...] = jnp.maximum(act(ymax), act(ymin))


def _edge_conv(gath, x, wpad_t, scale, beta, dp, o, nbat, ch=256):
    nb = _N // ch
    return pl.pallas_call(
        functools.partial(_conv_kernel, ch=ch, dp=dp, o=o),
        grid=(nbat, nb),
        in_specs=[
            pl.BlockSpec((1, _K, ch, dp), lambda b, j: (b, 0, j, 0)),
            pl.BlockSpec((1, ch, dp), lambda b, j: (b * nb + j, 0, 0)),
            pl.BlockSpec((2 * dp, o), lambda b, j: (0, 0)),
            pl.BlockSpec((1, o), lambda b, j: (0, 0)),
            pl.BlockSpec((1, o), lambda b, j: (0, 0)),
        ],
        out_specs=pl.BlockSpec((ch, o), lambda b, j: (b * nb + j, 0)),
        out_shape=jax.ShapeDtypeStruct((nbat * _N, o), jnp.float32),
    )(gath, x.reshape(nbat * nb, ch, dp), wpad_t, scale, beta)


def _pool_kernel(x1_ref, x2_ref, x3_ref, x4_ref, w_ref, s_ref, bt_ref,
                 out_ref):
    cat = jnp.concatenate(
        [x1_ref[...], x2_ref[...], x3_ref[...], x4_ref[...]], axis=1)
    y = lax.dot_general(cat.astype(jnp.bfloat16),
                        w_ref[...].astype(jnp.bfloat16),
                        (((1,), (0,)), ((), ())),
                        preferred_element_type=jnp.float32)
    t = y * s_ref[...] + bt_ref[...]
    z = jnp.where(t >= 0, t, 0.2 * t)
    out_ref[0] = jnp.max(z, axis=0, keepdims=True)


def _pool(x1, x2, x3, x4, w5_t, scale5, beta5, nbat):
    return pl.pallas_call(
        _pool_kernel,
        grid=(nbat,),
        in_specs=[
            pl.BlockSpec((_N, 64), lambda b: (b, 0)),
            pl.BlockSpec((_N, 64), lambda b: (b, 0)),
            pl.BlockSpec((_N, 128), lambda b: (b, 0)),
            pl.BlockSpec((_N, 256), lambda b: (b, 0)),
            pl.BlockSpec((512, 512), lambda b: (0, 0)),
            pl.BlockSpec((1, 512), lambda b: (0, 0)),
            pl.BlockSpec((1, 512), lambda b: (0, 0)),
        ],
        out_specs=pl.BlockSpec((1, 1, 512), lambda b: (b, 0, 0)),
        out_shape=jax.ShapeDtypeStruct((nbat, 1, 512), jnp.float32),
    )(x1, x2, x3, x4, w5_t, scale5, beta5)


def _mlp_kernel(p_ref, w1_ref, b1_ref, s1_ref, e1_ref, w2_ref, b2_ref,
                s2_ref, e2_ref, w3_ref, b3_ref, out_ref):
    def mm(a, w):
        return lax.dot_general(a.astype(jnp.bfloat16),
                               w[...].astype(jnp.bfloat16),
                               (((1,), (0,)), ((), ())),
                               preferred_element_type=jnp.float32)

    y = mm(p_ref[...], w1_ref) + b1_ref[...]
    t = y * s1_ref[...] + e1_ref[...]
    y = jnp.where(t >= 0, t, 0.2 * t)
    y = mm(y, w2_ref) + b2_ref[...]
    t = y * s2_ref[...] + e2_ref[...]
    y = jnp.where(t >= 0, t, 0.2 * t)
    out_ref[...] = mm(y, w3_ref) + b3_ref[...]


def _mlp(pooled, wc1_t, bc1, sc1, ec1, wc2_t, bc2, sc2, ec2, wc3_t, bc3):
    args = (pooled, wc1_t, bc1, sc1, ec1, wc2_t, bc2, sc2, ec2, wc3_t, bc3)
    return pl.pallas_call(
        _mlp_kernel,
        in_specs=[pl.BlockSpec(a.shape, lambda nd=len(a.shape): (0,) * nd)
                  for a in args],
        out_specs=pl.BlockSpec((_B, 40), lambda: (0, 0)),
        out_shape=jax.ShapeDtypeStruct((_B, 40), jnp.float32),
    )(*args)


def _layer(x_rows, dp, wpad_t, scale, beta, o, nbat):
    idx = _knn_topk(x_rows, dp, nbat)
    rows = _gather_rows(x_rows, idx.reshape(-1), dp)
    gath = rows.reshape(nbat, _K, _N, dp)
    return _edge_conv(gath, x_rows, wpad_t, scale, beta, dp, o, nbat)


def kernel(x, W1, g1, b1, W2, g2, b2, W3, g3, b3, W4, g4, b4, W5, g5, b5,
           Wc1, bc1, gc1, bec1, Wc2, bc2, gc2, bec2, Wc3, bc3):
    rsq = jnp.sqrt(jnp.float32(1.0) + _EPS)

    def row(v):
        return v.reshape(1, -1)

    def wpad(W, d):
        o = W.shape[0]
        wp = jnp.zeros((256, o), jnp.float32)
        return wp.at[0:d, :].set(W[:, 0:d].T).at[128:128 + d, :].set(
            W[:, d:2 * d].T)

    def pad128(v):
        return jnp.pad(v, ((0, 0), (0, 128 - v.shape[1])))

    nh = _B // 2

    def half(xh):
        x0p = pad128(xh.reshape(nh * _N, 3))
        x1 = _layer(x0p, 128, wpad(W1, 3), row(g1 / rsq), row(b1), 64, nh)
        x2 = _layer(pad128(x1), 128, wpad(W2, 64), row(g2 / rsq), row(b2),
                    64, nh)
        x3 = _layer(pad128(x2), 128, wpad(W3, 64), row(g3 / rsq), row(b3),
                    128, nh)
        x4 = _layer(x3, 128, W4.T, row(g4 / rsq), row(b4), 256, nh)
        return _pool(x1, x2, x3, x4, W5.T, row(g5 / rsq),
                     row(b5), nh).reshape(nh, 512)

    pooled = jnp.concatenate([half(x[:nh]), half(x[nh:])], axis=0)

    return _mlp(pooled, Wc1.T, row(bc1), row(gc1 / rsq), row(bec1),
                Wc2.T, row(bc2), row(gc2 / rsq), row(bec2),
                Wc3.T, row(bc3))


# speedup vs baseline: 20.6113x; 1.3879x over previous
import functools

import jax
import jax.numpy as jnp
from jax import lax
from jax.experimental import pallas as pl
from jax.experimental.pallas import tpu as pltpu
from jax.experimental.pallas import tpu_sc as plsc

_B = 16
_N = 1024
_K = 20
_EPS = 1e-5
_NEG = -3.0e38


def _knn_kernel(x_ref, idx_ref, tidx_ref, *, n, dp):
    b = pl.program_id(0)
    x = x_ref[...]
    xx = jnp.sum(x * x, axis=1, keepdims=True)
    xb = x.astype(jnp.bfloat16)
    g = lax.dot_general(xb, xb, (((1,), (1,)), ((), ())),
                        preferred_element_type=jnp.float32)
    inner = -2.0 * g
    xxr = jnp.transpose(xx)
    score = ((-xxr) - inner) - xx
    lane = lax.broadcasted_iota(jnp.int32, (n, n), 1)
    cur = score
    for i in range(_K):
        m = jnp.max(cur, axis=1, keepdims=True)
        cand = jnp.where(cur == m, lane, n + 1)
        am = jnp.min(cand, axis=1, keepdims=True)
        tidx_ref[:, pl.ds(i, 1)] = am
        cur = jnp.where(lane == am, _NEG, cur)
    t = jnp.transpose(tidx_ref[...])
    idx_ref[0] = t[:_K, :] + b * n


def _knn_topk(x, dp, nbat):
    return pl.pallas_call(
        functools.partial(_knn_kernel, n=_N, dp=dp),
        grid=(nbat,),
        in_specs=[pl.BlockSpec((_N, dp), lambda b: (b, 0))],
        out_specs=pl.BlockSpec((1, _K, _N), lambda b: (b, 0, 0)),
        out_shape=jax.ShapeDtypeStruct((nbat, _K, _N), jnp.int32),
        scratch_shapes=[pltpu.VMEM((_N, 32), jnp.int32)],
    )(x)


def _gather_rows(table, idx_flat, dp):
    m = idx_flat.shape[0]
    info = plsc.get_sparse_core_info()
    nw = info.num_cores * info.num_subcores
    chunk = m // nw
    win = 128
    nwin = chunk // win

    @functools.partial(
        pl.kernel,
        mesh=plsc.VectorSubcoreMesh(core_axis_name="c", subcore_axis_name="s"),
        out_type=jax.ShapeDtypeStruct((m, dp), jnp.float32),
        scratch_types=[
            pltpu.VMEM((2, win), jnp.int32),
            pltpu.VMEM((2, win, dp), jnp.float32),
            pltpu.SemaphoreType.DMA,
            pltpu.SemaphoreType.DMA,
        ],
    )
    def gk(table_hbm, idx_hbm, out_hbm, idx_v, rows_v, gsem, osem):
        wid = lax.axis_index("s") * info.num_cores + lax.axis_index("c")
        base = wid * chunk

        def body(w, carry):
            slot = lax.rem(w, 2)
            off = base + w * win
            pltpu.sync_copy(idx_hbm.at[pl.ds(off, win)], idx_v.at[slot])
            pltpu.async_copy(table_hbm.at[idx_v.at[slot]],
                             rows_v.at[slot], gsem).wait()
            pltpu.async_copy(rows_v.at[slot], out_hbm.at[pl.ds(off, win)],
                             osem)

            @pl.when(w >= 1)
            def _():
                pltpu.make_async_copy(
                    rows_v.at[slot], out_hbm.at[pl.ds(off, win)], osem
                ).wait()

            return carry

        lax.fori_loop(0, nwin, body, 0)
        pltpu.make_async_copy(rows_v.at[0],
                              out_hbm.at[pl.ds(base, win)], osem).wait()

    return gk(table, idx_flat)


def _conv_kernel(g_ref, x_ref, w_ref, s_ref, bt_ref, out_ref, *, ch, dp, o):
    xb = x_ref[...]
    g = g_ref[0]
    diff = g - xb
    xbk = jnp.broadcast_to(xb, (_K, ch, dp))
    f = jnp.concatenate([diff, xbk], axis=2)
    f2 = f.reshape(_K * ch, 2 * dp).astype(jnp.bfloat16)
    w = w_ref[...].astype(jnp.bfloat16)
    y = lax.dot_general(f2, w, (((1,), (0,)), ((), ())),
                        preferred_element_type=jnp.float32)
    ymax = y[0:ch, :]
    ymin = y[0:ch, :]
    for k in range(1, _K):
        blk = y[k * ch:(k + 1) * ch, :]
        ymax = jnp.maximum(ymax, blk)
        ymin = jnp.minimum(ymin, blk)
    s = s_ref[...]
    bt = bt_ref[...]

    def act(v):
        t = v * s + bt
        return jnp.where(t >= 0, t, 0.2 * t)

    out_ref[...] = jnp.maximum(act(ymax), act(ymin))


def _edge_conv(gath, x, wpad_t, scale, beta, dp, o, nbat, ch=256):
    nb = _N // ch
    return pl.pallas_call(
        functools.partial(_conv_kernel, ch=ch, dp=dp, o=o),
        grid=(nbat, nb),
        in_specs=[
            pl.BlockSpec((1, _K, ch, dp), lambda b, j: (b, 0, j, 0)),
            pl.BlockSpec((1, ch, dp), lambda b, j: (b * nb + j, 0, 0)),
            pl.BlockSpec((2 * dp, o), lambda b, j: (0, 0)),
            pl.BlockSpec((1, o), lambda b, j: (0, 0)),
            pl.BlockSpec((1, o), lambda b, j: (0, 0)),
        ],
        out_specs=pl.BlockSpec((ch, o), lambda b, j: (b * nb + j, 0)),
        out_shape=jax.ShapeDtypeStruct((nbat * _N, o), jnp.float32),
    )(gath, x.reshape(nbat * nb, ch, dp), wpad_t, scale, beta)


def _pool_kernel(x1_ref, x2_ref, x3_ref, x4_ref, w_ref, s_ref, bt_ref,
                 out_ref):
    cat = jnp.concatenate(
        [x1_ref[...], x2_ref[...], x3_ref[...], x4_ref[...]], axis=1)
    y = lax.dot_general(cat.astype(jnp.bfloat16),
                        w_ref[...].astype(jnp.bfloat16),
                        (((1,), (0,)), ((), ())),
                        preferred_element_type=jnp.float32)
    t = y * s_ref[...] + bt_ref[...]
    z = jnp.where(t >= 0, t, 0.2 * t)
    out_ref[0] = jnp.max(z, axis=0, keepdims=True)


def _pool(x1, x2, x3, x4, w5_t, scale5, beta5, nbat):
    return pl.pallas_call(
        _pool_kernel,
        grid=(nbat,),
        in_specs=[
            pl.BlockSpec((_N, 64), lambda b: (b, 0)),
            pl.BlockSpec((_N, 64), lambda b: (b, 0)),
            pl.BlockSpec((_N, 128), lambda b: (b, 0)),
            pl.BlockSpec((_N, 256), lambda b: (b, 0)),
            pl.BlockSpec((512, 512), lambda b: (0, 0)),
            pl.BlockSpec((1, 512), lambda b: (0, 0)),
            pl.BlockSpec((1, 512), lambda b: (0, 0)),
        ],
        out_specs=pl.BlockSpec((1, 1, 512), lambda b: (b, 0, 0)),
        out_shape=jax.ShapeDtypeStruct((nbat, 1, 512), jnp.float32),
    )(x1, x2, x3, x4, w5_t, scale5, beta5)


def _mlp_kernel(p_ref, w1_ref, b1_ref, s1_ref, e1_ref, w2_ref, b2_ref,
                s2_ref, e2_ref, w3_ref, b3_ref, out_ref):
    def mm(a, w):
        return lax.dot_general(a.astype(jnp.bfloat16),
                               w[...].astype(jnp.bfloat16),
                               (((1,), (0,)), ((), ())),
                               preferred_element_type=jnp.float32)

    y = mm(p_ref[...], w1_ref) + b1_ref[...]
    t = y * s1_ref[...] + e1_ref[...]
    y = jnp.where(t >= 0, t, 0.2 * t)
    y = mm(y, w2_ref) + b2_ref[...]
    t = y * s2_ref[...] + e2_ref[...]
    y = jnp.where(t >= 0, t, 0.2 * t)
    out_ref[...] = mm(y, w3_ref) + b3_ref[...]


def _mlp(pooled, wc1_t, bc1, sc1, ec1, wc2_t, bc2, sc2, ec2, wc3_t, bc3):
    args = (pooled, wc1_t, bc1, sc1, ec1, wc2_t, bc2, sc2, ec2, wc3_t, bc3)
    return pl.pallas_call(
        _mlp_kernel,
        in_specs=[pl.BlockSpec(a.shape, lambda nd=len(a.shape): (0,) * nd)
                  for a in args],
        out_specs=pl.BlockSpec((_B, 40), lambda: (0, 0)),
        out_shape=jax.ShapeDtypeStruct((_B, 40), jnp.float32),
    )(*args)


def _layer(x_rows, dp, wpad_t, scale, beta, o, nbat):
    idx = _knn_topk(x_rows, dp, nbat)
    rows = _gather_rows(x_rows, idx.reshape(-1), dp)
    gath = rows.reshape(nbat, _K, _N, dp)
    return _edge_conv(gath, x_rows, wpad_t, scale, beta, dp, o, nbat)


def _shard_map(f, mesh, in_specs, out_specs):
    sm = getattr(jax, "shard_map", None)
    if sm is None:
        from jax.experimental.shard_map import shard_map as sm
    try:
        return sm(f, mesh=mesh, in_specs=in_specs, out_specs=out_specs,
                  check_vma=False)
    except TypeError:
        return sm(f, mesh=mesh, in_specs=in_specs, out_specs=out_specs,
                  check_rep=False)


def kernel(x, W1, g1, b1, W2, g2, b2, W3, g3, b3, W4, g4, b4, W5, g5, b5,
           Wc1, bc1, gc1, bec1, Wc2, bc2, gc2, bec2, Wc3, bc3):
    rsq = jnp.sqrt(jnp.float32(1.0) + _EPS)

    def row(v):
        return v.reshape(1, -1)

    def wpad(W, d):
        o = W.shape[0]
        wp = jnp.zeros((256, o), jnp.float32)
        return wp.at[0:d, :].set(W[:, 0:d].T).at[128:128 + d, :].set(
            W[:, d:2 * d].T)

    def pad128(v):
        return jnp.pad(v, ((0, 0), (0, 128 - v.shape[1])))

    ws = (W1, g1, b1, W2, g2, b2, W3, g3, b3, W4, g4, b4, W5, g5, b5,
          Wc1, bc1, gc1, bec1, Wc2, bc2, gc2, bec2, Wc3, bc3)

    def local_fn(xl, w1, a1, c1, w2, a2, c2, w3, a3, c3, w4, a4, c4,
                 w5, a5, c5, wc1, cc1, ac1, ec1, wc2, cc2, ac2, ec2,
                 wc3, cc3):
        def chain(xh, nbat):
            x0p = pad128(xh.reshape(nbat * _N, 3))
            x1 = _layer(x0p, 128, wpad(w1, 3), row(a1 / rsq), row(c1),
                        64, nbat)
            x2 = _layer(pad128(x1), 128, wpad(w2, 64), row(a2 / rsq),
                        row(c2), 64, nbat)
            x3 = _layer(pad128(x2), 128, wpad(w3, 64), row(a3 / rsq),
                        row(c3), 128, nbat)
            x4 = _layer(x3, 128, w4.T, row(a4 / rsq), row(c4), 256, nbat)
            return _pool(x1, x2, x3, x4, w5.T, row(a5 / rsq),
                         row(c5), nbat).reshape(nbat, 512)

        nq = xl.shape[0] // 2
        pooled = jnp.concatenate(
            [chain(xl[:nq], nq), chain(xl[nq:], nq)], axis=0)
        pooled = lax.all_gather(pooled, "d", axis=0, tiled=True)
        return _mlp(pooled, wc1.T, row(cc1), row(ac1 / rsq), row(ec1),
                    wc2.T, row(cc2), row(ac2 / rsq), row(ec2),
                    wc3.T, row(cc3))

    ndev = 2 if len(jax.devices()) >= 2 else 1
    mesh = jax.make_mesh((ndev,), ("d",))
    P = jax.sharding.PartitionSpec
    ns = jax.sharding.NamedSharding
    x = jax.reshard(x, ns(mesh, P("d", None, None)))
    ws = tuple(jax.reshard(w, ns(mesh, P())) for w in ws)
    specs = (P("d"),) + tuple(P() for _ in ws)
    return _shard_map(local_fn, mesh, specs, P())(x, *ws)
